# Initial kernel scaffold; baseline (speedup 1.0000x reference)
import jax
import jax.numpy as jnp
from jax.experimental import pallas as pl


def kernel(h0, h1, h2, h3, Wg, bg):
    raise NotImplementedError("write your pallas kernel here")


# speedup vs baseline: 14.1427x; 14.1427x over previous
import jax
import jax.numpy as jnp
from jax.experimental import pallas as pl
from jax.experimental.pallas import tpu as pltpu

T, B, D, L = 2048, 4, 1024, 4
TB = 128


def _gate_kernel(h0_ref, h1_ref, h2_ref, h3_ref, wg_ref, w_ref, acc_ref):
    i = pl.program_id(0)
    nsteps = pl.num_programs(0)

    @pl.when(i == 0)
    def _init():
        acc_ref[...] = jnp.zeros_like(acc_ref)

    wgv = wg_ref[...][None]
    for l, h_ref in enumerate((h0_ref, h1_ref, h2_ref, h3_ref)):
        hb = h_ref[...]
        acc_ref[l, 0:B] += jnp.sum(hb * wgv, axis=(0, 2))

    @pl.when(i == nsteps - 1)
    def _finish():
        g = acc_ref[...] * (1.0 / T)
        rows = jax.lax.broadcasted_iota(jnp.int32, g.shape, 0)
        neg = jnp.float32(-jnp.inf)
        g = jnp.where(rows < L, g, neg)
        m1 = jnp.max(g, axis=0, keepdims=True)
        i1 = jnp.min(jnp.where(g == m1, rows, L + 4), axis=0, keepdims=True)
        mask1 = rows == i1
        g2 = jnp.where(mask1, neg, g)
        m2 = jnp.max(g2, axis=0, keepdims=True)
        i2 = jnp.min(jnp.where(g2 == m2, rows, L + 4), axis=0, keepdims=True)
        mask2 = rows == i2
        e2 = jnp.exp(m2 - m1)
        w1 = 0.5 / (1.0 + e2)
        w2 = (0.5 * e2) / (1.0 + e2)
        w_ref[...] = jnp.where(mask1, w1, 0.0) + jnp.where(mask2, w2, 0.0)


def _combine_kernel(wt_ref, h0_ref, h1_ref, h2_ref, h3_ref, out_ref):
    acc = None
    for l, h_ref in enumerate((h0_ref, h1_ref, h2_ref, h3_ref)):
        wl = wt_ref[0:B, l : l + 1][None]
        term = h_ref[...] * wl
        acc = term if acc is None else acc + term
    out_ref[...] = acc


def kernel(h0, h1, h2, h3, Wg, bg):
    del bg
    wg2 = Wg.reshape(1, D)
    h_spec = pl.BlockSpec((TB, B, D), lambda i: (i, 0, 0))
    w = pl.pallas_call(
        _gate_kernel,
        grid=(T // TB,),
        in_specs=[h_spec, h_spec, h_spec, h_spec,
                  pl.BlockSpec((1, D), lambda i: (0, 0))],
        out_specs=pl.BlockSpec((8, 128), lambda i: (0, 0)),
        out_shape=jax.ShapeDtypeStruct((8, 128), jnp.float32),
        scratch_shapes=[pltpu.VMEM((8, 128), jnp.float32)],
    )(h0, h1, h2, h3, wg2)
    wt = w.T
    out = pl.pallas_call(
        _combine_kernel,
        grid=(T // TB,),
        in_specs=[pl.BlockSpec((128, 8), lambda i: (0, 0)),
                  h_spec, h_spec, h_spec, h_spec],
        out_specs=h_spec,
        out_shape=jax.ShapeDtypeStruct((T, B, D), jnp.float32),
    )(wt, h0, h1, h2, h3)
    return out
